# Initial kernel scaffold; baseline (speedup 1.0000x reference)
import jax
import jax.numpy as jnp
from jax.experimental import pallas as pl


def kernel(x, edge_index, edge, W1c, b1c, W2c, b2c, W3c, b3c, W1j, b1j, W2j, b2j, Wl, bl, beta):
    raise NotImplementedError("write your pallas kernel here")


# speedup vs baseline: 3.4303x; 3.4303x over previous
import jax
import jax.numpy as jnp
from jax import lax
from jax.experimental import pallas as pl
from jax.experimental.pallas import tpu as pltpu
from jax.experimental.pallas import tpu_sc as plsc

NN = 10000
WRD = 320
WRDP = 384
E_PAD = 160256
EPT = E_PAD // 16
QROWS = 2500
RPT = QROWS * WRD // 16
ZB = 10000
QCH = 64


def _sc_build_body(src_hbm, dst_hbm, msk_hbm, out_hbm,
                   srcb, dstb, mskb, idxb, valb, zb, acc):
    c = lax.axis_index("c")
    s = lax.axis_index("s")

    base = s * EPT
    pltpu.sync_copy(src_hbm.at[pl.ds(base, EPT)], srcb)
    pltpu.sync_copy(dst_hbm.at[pl.ds(base, EPT)], dstb)
    pltpu.sync_copy(msk_hbm.at[pl.ds(base, EPT)], mskb)

    def zloop(i, carry):
        zb[pl.ds(i * 16, 16)] = jnp.zeros((16,), jnp.int32)
        return carry

    for p in range(2):
        lo = p * 2 * QROWS + c * QROWS
        lax.fori_loop(0, ZB // 16, zloop, 0)
        for q in range(RPT // ZB):
            pltpu.sync_copy(zb, acc.at[pl.ds(s * RPT + q * ZB, ZB)])
        plsc.subcore_barrier()

        def eloop(i, carry):
            sl = pl.ds(i * 16, 16)
            sv = srcb[sl]
            dv = dstb[sl]
            mv = mskb[sl]
            word = jnp.right_shift(dv, 5)
            bit = jnp.bitwise_and(dv, 31)
            val = jnp.left_shift(mv, bit)
            rel = sv - lo
            inh = (rel >= 0) & (rel < QROWS)
            idxb[sl] = jnp.where(inh, rel * WRD + word, 0)
            valb[sl] = jnp.where(inh, val, 0)
            return carry
        lax.fori_loop(0, EPT // 16, eloop, 0)

        pltpu.sync_copy(valb, acc.at[idxb], add=True)
        plsc.subcore_barrier()
        for q in range(RPT // ZB):
            pltpu.sync_copy(acc.at[pl.ds(s * RPT + q * ZB, ZB)], zb)
            pltpu.sync_copy(
                zb, out_hbm.at[pl.ds(lo * WRD + s * RPT + q * ZB, ZB)])


def _sc_gather_body(a_hbm, x_hbm, ei_hbm, ej_hbm, cn_hbm, xij_hbm,
                    ii, jj, ab, bb, xa, xb, sem):
    c = lax.axis_index("c")
    s = lax.axis_index("s")
    wid = c * 16 + s
    for h in range(2):
        base = wid * (2 * QCH) + h * QCH
        pltpu.sync_copy(ei_hbm.at[pl.ds(base, QCH)], ii)
        pltpu.sync_copy(ej_hbm.at[pl.ds(base, QCH)], jj)
        pltpu.async_copy(a_hbm.at[ii], ab, sem).wait()
        pltpu.async_copy(a_hbm.at[jj], bb, sem).wait()
        pltpu.async_copy(x_hbm.at[ii], xa, sem).wait()
        pltpu.async_copy(x_hbm.at[jj], xb, sem).wait()

        def rloop(r, carry):
            for wck in range(WRD // 16):
                slc = (r, pl.ds(wck * 16, 16))
                ab[slc] = jnp.bitwise_and(ab[slc], bb[slc])
            for xc in range(128 // 16):
                slx = (r, pl.ds(xc * 16, 16))
                xa[slx] = xa[slx] * xb[slx]
            return carry
        lax.fori_loop(0, QCH, rloop, 0)
        pltpu.sync_copy(ab, cn_hbm.at[pl.ds(base, QCH)])
        pltpu.sync_copy(xa, xij_hbm.at[pl.ds(base, QCH)])


def _tc_body(cn_ref, xre_ref, xij_ref,
             w1c, c1, w2c, c2, w3c, c3, w1j, j1, w2j, j2, wl, cl, bt,
             out_ref):
    cnw = cn_ref[...]
    acc = jnp.zeros((cnw.shape[0], 128), jnp.float32)
    for b in range(32):
        u = jnp.bitwise_and(jnp.right_shift(cnw, b), 1).astype(jnp.float32)
        acc = acc + jnp.dot(u, xre_ref[b], preferred_element_type=jnp.float32)
    h = jnp.maximum(jnp.dot(acc, w1c[...], preferred_element_type=jnp.float32) + c1[...], 0.0)
    h = jnp.maximum(jnp.dot(h, w2c[...], preferred_element_type=jnp.float32) + c2[...], 0.0)
    hcn = jnp.dot(h, w3c[...], preferred_element_type=jnp.float32) + c3[...]
    xij = xij_ref[...]
    hj = jnp.maximum(jnp.dot(xij, w1j[...], preferred_element_type=jnp.float32) + j1[...], 0.0)
    hij = jnp.dot(hj, w2j[...], preferred_element_type=jnp.float32) + j2[...]
    z = hcn * bt[...] + hij
    out_ref[...] = jnp.dot(z, wl[...], preferred_element_type=jnp.float32) + cl[...]


def kernel(x, edge_index, edge, W1c, b1c, W2c, b2c, W3c, b3c,
           W1j, b1j, W2j, b2j, Wl, bl, beta):
    n = x.shape[0]
    e = edge_index.shape[1]
    eq = edge.shape[0]

    key = edge_index[0] * n + edge_index[1]
    skey = jnp.sort(key)
    m = jnp.concatenate([jnp.ones((1,), jnp.int32),
                         (skey[1:] != skey[:-1]).astype(jnp.int32)])
    sd = skey // n
    dd = skey % n
    pad = E_PAD - e
    src_p = jnp.pad(sd, (0, pad))
    dst_p = jnp.pad(dd, (0, pad))
    msk_p = jnp.pad(m, (0, pad))

    mesh = plsc.VectorSubcoreMesh(core_axis_name="c", subcore_axis_name="s")
    a_flat = pl.kernel(
        _sc_build_body, mesh=mesh,
        out_type=jax.ShapeDtypeStruct((NN * WRD,), jnp.int32),
        scratch_types=[
            pltpu.VMEM((EPT,), jnp.int32),
            pltpu.VMEM((EPT,), jnp.int32),
            pltpu.VMEM((EPT,), jnp.int32),
            pltpu.VMEM((EPT,), jnp.int32),
            pltpu.VMEM((EPT,), jnp.int32),
            pltpu.VMEM((ZB,), jnp.int32),
            pltpu.VMEM_SHARED((QROWS * WRD,), jnp.int32),
        ],
    )(src_p, dst_p, msk_p)
    a2 = a_flat.reshape(NN, WRD)

    ei = edge[:, 0]
    ej = edge[:, 1]
    cn, xij = pl.kernel(
        _sc_gather_body, mesh=mesh,
        compiler_params=pltpu.CompilerParams(use_tc_tiling_on_sc=False),
        out_type=[jax.ShapeDtypeStruct((eq, WRD), jnp.int32),
                  jax.ShapeDtypeStruct((eq, 128), jnp.float32)],
        scratch_types=[
            pltpu.VMEM((QCH,), jnp.int32),
            pltpu.VMEM((QCH,), jnp.int32),
            pltpu.VMEM((QCH, WRD), jnp.int32),
            pltpu.VMEM((QCH, WRD), jnp.int32),
            pltpu.VMEM((QCH, 128), jnp.float32),
            pltpu.VMEM((QCH, 128), jnp.float32),
            pltpu.SemaphoreType.DMA,
        ],
    )(a2, x, ei, ej)

    cn_p = jnp.pad(cn, ((0, 0), (0, WRDP - WRD)))
    x_re = jnp.pad(x, ((0, 32 * WRDP - n), (0, 0))).reshape(WRDP, 32, 128)
    x_re = x_re.transpose(1, 0, 2)

    bq = 512
    grid = eq // bq
    full = lambda i: (0, 0)
    out = pl.pallas_call(
        _tc_body,
        grid=(grid,),
        in_specs=[
            pl.BlockSpec((bq, WRDP), lambda i: (i, 0)),
            pl.BlockSpec((32, WRDP, 128), lambda i: (0, 0, 0)),
            pl.BlockSpec((bq, 128), lambda i: (i, 0)),
            pl.BlockSpec(W1c.shape, full), pl.BlockSpec((1, 256), full),
            pl.BlockSpec(W2c.shape, full), pl.BlockSpec((1, 256), full),
            pl.BlockSpec(W3c.shape, full), pl.BlockSpec((1, 256), full),
            pl.BlockSpec(W1j.shape, full), pl.BlockSpec((1, 256), full),
            pl.BlockSpec(W2j.shape, full), pl.BlockSpec((1, 256), full),
            pl.BlockSpec(Wl.shape, full), pl.BlockSpec((1, 1), full),
            pl.BlockSpec((1, 1), full),
        ],
        out_specs=pl.BlockSpec((bq, 1), lambda i: (i, 0)),
        out_shape=jax.ShapeDtypeStruct((eq, 1), jnp.float32),
    )(cn_p, x_re, xij,
      W1c, b1c.reshape(1, -1), W2c, b2c.reshape(1, -1),
      W3c, b3c.reshape(1, -1), W1j, b1j.reshape(1, -1),
      W2j, b2j.reshape(1, -1), Wl, bl.reshape(1, 1), beta.reshape(1, 1))
    return out
